# baseline (speedup 1.0000x reference)
import dataclasses

import jax
import jax.numpy as jnp
from jax import lax
from jax.experimental import pallas as pl
from jax.experimental.pallas import tpu as pltpu
from jax.experimental.pallas import tpu_sc as plsc

_NUM_CORES = 2
_NUM_SUBCORES = 16
_NUM_WORKERS = _NUM_CORES * _NUM_SUBCORES
_NB = 16
_LANES = 16


def kernel(ids, table):
    batch, hist = ids.shape
    vocab, d = table.shape
    batches_per_worker = batch // _NUM_WORKERS
    hist_pad = (hist + 7) // 8 * 8
    chunk = _NB * hist_pad

    mesh = plsc.VectorSubcoreMesh(core_axis_name="c", subcore_axis_name="s")
    cp = dataclasses.replace(pltpu.CompilerParams(), use_tc_tiling_on_sc=False)

    @pl.kernel(
        out_type=jax.ShapeDtypeStruct((batch, hist, d), table.dtype),
        mesh=mesh,
        scratch_types=[
            pltpu.VMEM((chunk,), jnp.int32),
            pltpu.VMEM((chunk, d), table.dtype),
            pltpu.SemaphoreType.DMA,
        ],
        compiler_params=cp,
    )
    def gather_kernel(table_hbm, ids_hbm, out_hbm, idx_v, rows_v, sem):
        wid = lax.axis_index("s") * _NUM_CORES + lax.axis_index("c")
        b_base = wid * batches_per_worker

        @pl.loop(0, chunk, step=_LANES)
        def _(i):
            idx_v.at[pl.ds(i, _LANES)][...] = jnp.zeros((_LANES,), jnp.int32)

        @pl.loop(0, batches_per_worker, step=_NB)
        def _(boff):
            for b in range(_NB):
                pltpu.sync_copy(ids_hbm.at[b_base + boff + b],
                                idx_v.at[pl.ds(b * hist_pad, hist)])
            pltpu.async_copy(table_hbm.at[idx_v], rows_v, sem).wait()
            for b in range(_NB):
                pltpu.sync_copy(rows_v.at[pl.ds(b * hist_pad, hist), :],
                                out_hbm.at[b_base + boff + b])

    return gather_kernel(table, ids)


# speedup vs baseline: 1.6546x; 1.6546x over previous
import dataclasses

import jax
import jax.numpy as jnp
from jax import lax
from jax.experimental import pallas as pl
from jax.experimental.pallas import tpu as pltpu
from jax.experimental.pallas import tpu_sc as plsc

_NUM_CORES = 2
_NUM_SUBCORES = 16
_NUM_WORKERS = _NUM_CORES * _NUM_SUBCORES
_CHUNK = 400
_LANES = 16


def _flatten_ids(ids):
    batch, hist = ids.shape
    n = batch * hist
    per_worker = n // _NUM_WORKERS
    flat = ids.reshape(n)

    mesh = plsc.VectorSubcoreMesh(core_axis_name="c", subcore_axis_name="s")

    @pl.kernel(
        out_type=jax.ShapeDtypeStruct((n,), jnp.int32),
        mesh=mesh,
        scratch_types=[pltpu.VMEM((per_worker,), jnp.int32)],
    )
    def copy_kernel(in_hbm, out_hbm, buf_v):
        wid = lax.axis_index("s") * _NUM_CORES + lax.axis_index("c")
        sl = pl.ds(wid * per_worker, per_worker)
        pltpu.sync_copy(in_hbm.at[sl], buf_v)
        pltpu.sync_copy(buf_v, out_hbm.at[sl])

    return copy_kernel(flat)


def kernel(ids, table):
    batch, hist = ids.shape
    vocab, d = table.shape
    num_indices = batch * hist
    per_worker = num_indices // _NUM_WORKERS

    flat = _flatten_ids(ids)

    mesh = plsc.VectorSubcoreMesh(core_axis_name="c", subcore_axis_name="s")
    cp = dataclasses.replace(pltpu.CompilerParams(), use_tc_tiling_on_sc=False)

    @pl.kernel(
        out_type=jax.ShapeDtypeStruct((batch, hist, d), table.dtype),
        mesh=mesh,
        scratch_types=[
            pltpu.VMEM((_CHUNK,), jnp.int32),
            pltpu.VMEM((_CHUNK, d), table.dtype),
            pltpu.SemaphoreType.DMA,
        ],
        compiler_params=cp,
    )
    def gather_kernel(table_hbm, ids_hbm, out_hbm, idx_v, rows_v, sem):
        wid = lax.axis_index("s") * _NUM_CORES + lax.axis_index("c")
        base = wid * per_worker
        b_base = wid * (per_worker // hist)
        nb = _CHUNK // hist

        @pl.loop(0, per_worker, step=_CHUNK)
        def _(off):
            pltpu.sync_copy(ids_hbm.at[pl.ds(base + off, _CHUNK)], idx_v)
            pltpu.async_copy(table_hbm.at[idx_v], rows_v, sem).wait()
            for b in range(nb):
                pltpu.sync_copy(rows_v.at[pl.ds(b * hist, hist), :],
                                out_hbm.at[b_base + off // hist + b])

    return gather_kernel(table, flat)
